# baseline (speedup 1.0000x reference)
import functools

import jax
import jax.numpy as jnp
from jax import lax
from jax.experimental import pallas as pl
from jax.experimental.pallas import tpu as pltpu
from jax.experimental.pallas import tpu_sc as plsc

N = 10000
NP = 10240
D = 256
H = 256
C = 40
E = 160000
B = 128
E2 = 163840
PAD = E2 - E
ROWS = E2 // B
HR = ROWS // 32
MR = ROWS // 16
SLAB = NP // 16
HALF = H // 2

_mesh = plsc.VectorSubcoreMesh(core_axis_name="c", subcore_axis_name="s")


@functools.partial(
    pl.kernel,
    out_type=jax.ShapeDtypeStruct((2 * NP,), jnp.float32),
    mesh=_mesh,
    scratch_types=[
        pltpu.VMEM_SHARED((NP,), jnp.float32),
        pltpu.VMEM((SLAB,), jnp.float32),
        pltpu.VMEM((B,), jnp.float32),
        pltpu.VMEM((HR, B), jnp.int32),
        [pltpu.SemaphoreType.DMA] * 4,
    ],
)
def _hist_k(dst_hbm, degp_hbm, hist, zbuf, obuf, iall, hsems):
    c = lax.axis_index("c")
    s = lax.axis_index("s")
    wid = s * 2 + c
    zero16 = jnp.zeros((16,), jnp.float32)
    one16 = jnp.ones((16,), jnp.float32)
    for i in range(SLAB // 16):
        zbuf[pl.ds(i * 16, 16)] = zero16
    for i in range(B // 16):
        obuf[pl.ds(i * 16, 16)] = one16
    pltpu.sync_copy(zbuf, hist.at[pl.ds(s * SLAB, SLAB)])
    pltpu.sync_copy(dst_hbm.at[pl.ds(wid * HR, HR)], iall)
    plsc.subcore_barrier()

    def body(t, carry):
        for b in range(4):
            j = 4 * t + b

            @pl.when(t > 0)
            def _():
                pltpu.make_async_copy(obuf, hist.at[iall.at[0]],
                                      hsems[b]).wait()

            pltpu.async_copy(obuf, hist.at[iall.at[j]], hsems[b], add=True)
        return carry

    lax.fori_loop(0, HR // 4, body, 0)
    for b in range(4):
        pltpu.make_async_copy(obuf, hist.at[iall.at[0]], hsems[b]).wait()
    plsc.subcore_barrier()
    pltpu.sync_copy(hist.at[pl.ds(s * SLAB, SLAB)],
                    degp_hbm.at[pl.ds(c * NP + s * SLAB, SLAB)])


_NBUF = 2
_CH = MR // 2


@functools.partial(
    pl.kernel,
    out_type=jax.ShapeDtypeStruct((2 * NP, HALF), jnp.float32),
    mesh=_mesh,
    scratch_types=[
        pltpu.VMEM_SHARED((NP, HALF), jnp.float32),
        pltpu.VMEM((_CH, B), jnp.int32),
        pltpu.VMEM((_CH, B), jnp.int32),
        pltpu.VMEM((_NBUF, B, HALF), jnp.float32),
        [pltpu.SemaphoreType.DMA] * _NBUF,
        [pltpu.SemaphoreType.DMA] * _NBUF,
    ],
)
def _gs_k(y_hbm, src_hbm, dst_hbm, out_hbm, acc, sall, dall, rbuf,
          gsems, ssems):
    c = lax.axis_index("c")
    s = lax.axis_index("s")
    base = s * MR
    off = jnp.zeros((16,), jnp.int32) + c * NP

    def stage_idx(k):
        pltpu.sync_copy(src_hbm.at[pl.ds(base + k * _CH, _CH)], sall)
        pltpu.sync_copy(dst_hbm.at[pl.ds(base + k * _CH, _CH)], dall)

        def adj(r, carry):
            for i in range(B // 16):
                sall[r, pl.ds(i * 16, 16)] = sall[r, pl.ds(i * 16, 16)] + off
            return carry

        lax.fori_loop(0, _CH, adj, 0)

    def start_gather(b, j):
        pltpu.async_copy(y_hbm.at[sall.at[j]], rbuf.at[b], gsems[b])

    def wait_gather(b):
        pltpu.make_async_copy(y_hbm.at[sall.at[0]], rbuf.at[b],
                              gsems[b]).wait()

    def start_scatter(b, j):
        pltpu.async_copy(rbuf.at[b], acc.at[dall.at[j]], ssems[b], add=True)

    def wait_scatter(b):
        pltpu.make_async_copy(rbuf.at[b], acc.at[dall.at[0]],
                              ssems[b]).wait()

    stage_idx(0)
    pltpu.sync_copy(y_hbm.at[pl.ds(c * NP + s * SLAB, SLAB)],
                    acc.at[pl.ds(s * SLAB, SLAB)])
    plsc.subcore_barrier()

    for k in range(MR // _CH):
        for b in range(_NBUF):
            start_gather(b, b)

        def body(t, carry):
            for b in range(_NBUF):
                j = _NBUF * t + b
                wait_gather(b)
                start_scatter(b, j)

                @pl.when(j < _CH - _NBUF)
                def _():
                    wait_scatter(b)
                    start_gather(b, j + _NBUF)

            return carry

        lax.fori_loop(0, _CH // _NBUF, body, 0)
        for b in range(_NBUF):
            wait_scatter(b)
        if k + 1 < MR // _CH:
            stage_idx(k + 1)

    plsc.subcore_barrier()
    pltpu.sync_copy(acc.at[pl.ds(s * SLAB, SLAB)],
                    out_hbm.at[pl.ds(c * NP + s * SLAB, SLAB)])


def _mm_scale_body(x_ref, w_ref, degp_ref, y_ref):
    d = degp_ref[...]
    deg = d[:, 0:1] + d[:, 1:2] + 1.0
    dis = lax.rsqrt(deg)
    xw = jnp.dot(x_ref[...], w_ref[...],
                 preferred_element_type=jnp.float32)
    y = xw * dis
    y_ref[0, :, :] = y[:, :HALF]
    y_ref[1, :, :] = y[:, HALF:]


def _epi_body(acc_ref, degp_ref, b0_ref, g_ref, be_ref, wc_ref, bc_ref, out_ref):
    d = degp_ref[...]
    deg = d[:, 0:1] + d[:, 1:2] + 1.0
    dis = lax.rsqrt(deg)
    h0 = jnp.maximum(acc_ref[0] * dis + b0_ref[:, :HALF], 0.0)
    h1 = jnp.maximum(acc_ref[1] * dis + b0_ref[:, HALF:], 0.0)
    mu = (jnp.sum(h0, axis=-1, keepdims=True)
          + jnp.sum(h1, axis=-1, keepdims=True)) * (1.0 / H)
    c0 = h0 - mu
    c1 = h1 - mu
    var = (jnp.sum(c0 * c0, axis=-1, keepdims=True)
           + jnp.sum(c1 * c1, axis=-1, keepdims=True)) * (1.0 / H)
    r = lax.rsqrt(var + 1e-5)
    n0 = c0 * r * g_ref[:, :HALF] + be_ref[:, :HALF]
    n1 = c1 * r * g_ref[:, HALF:] + be_ref[:, HALF:]
    out_ref[...] = (jnp.dot(n0, wc_ref[:HALF, :],
                            preferred_element_type=jnp.float32)
                    + jnp.dot(n1, wc_ref[HALF:, :],
                              preferred_element_type=jnp.float32)
                    + bc_ref[...])


_BN = 1000

_mm_call = pl.pallas_call(
    _mm_scale_body,
    grid=(N // _BN,),
    in_specs=[
        pl.BlockSpec((_BN, D), lambda j: (j, 0)),
        pl.BlockSpec((D, H), lambda j: (0, 0)),
        pl.BlockSpec((_BN, 2), lambda j: (j, 0)),
    ],
    out_specs=pl.BlockSpec((2, _BN, HALF), lambda j: (0, j, 0)),
    out_shape=jax.ShapeDtypeStruct((2, NP, HALF), jnp.float32),
)

_epi_call = pl.pallas_call(
    _epi_body,
    grid=(N // _BN,),
    in_specs=[
        pl.BlockSpec((2, _BN, HALF), lambda j: (0, j, 0)),
        pl.BlockSpec((_BN, 2), lambda j: (j, 0)),
        pl.BlockSpec((1, H), lambda j: (0, 0)),
        pl.BlockSpec((1, H), lambda j: (0, 0)),
        pl.BlockSpec((1, H), lambda j: (0, 0)),
        pl.BlockSpec((H, C), lambda j: (0, 0)),
        pl.BlockSpec((1, C), lambda j: (0, 0)),
    ],
    out_specs=pl.BlockSpec((_BN, C), lambda j: (j, 0)),
    out_shape=jax.ShapeDtypeStruct((N, C), jnp.float32),
)


def kernel(x, edge_index, W0, b0, gamma0, beta0, Wc, bc):
    src = edge_index[0]
    dst = edge_index[1]
    src_p = jnp.concatenate([src, jnp.zeros((PAD,), jnp.int32)])
    dst_p = jnp.concatenate([dst, jnp.full((PAD,), N, jnp.int32)])
    dst2d = dst_p.reshape(ROWS, B)
    src2d = src_p.reshape(ROWS, B)

    degp_flat = _hist_k(dst2d)
    degp = jnp.stack([degp_flat[:NP], degp_flat[NP:]], axis=1)
    yst = _mm_call(x, W0, degp)
    y2 = yst.reshape(2 * NP, HALF)

    accf = _gs_k(y2, src2d, dst2d)
    acc3 = accf.reshape(2, NP, HALF)

    return _epi_call(acc3, degp,
                     b0.reshape(1, H), gamma0.reshape(1, H),
                     beta0.reshape(1, H), Wc, bc.reshape(1, C))


# speedup vs baseline: 1.0011x; 1.0011x over previous
import functools

import jax
import jax.numpy as jnp
from jax import lax
from jax.experimental import pallas as pl
from jax.experimental.pallas import tpu as pltpu
from jax.experimental.pallas import tpu_sc as plsc

N = 10000
NP = 10240
D = 256
H = 256
C = 40
E = 160000
B = 128
E2 = 163840
PAD = E2 - E
ROWS = E2 // B
HR = ROWS // 32
MR = ROWS // 16
SLAB = NP // 16
HALF = H // 2

_mesh = plsc.VectorSubcoreMesh(core_axis_name="c", subcore_axis_name="s")


@functools.partial(
    pl.kernel,
    out_type=jax.ShapeDtypeStruct((2 * NP,), jnp.float32),
    mesh=_mesh,
    scratch_types=[
        pltpu.VMEM_SHARED((NP,), jnp.float32),
        pltpu.VMEM((SLAB,), jnp.float32),
        pltpu.VMEM((B,), jnp.float32),
        pltpu.VMEM((HR, B), jnp.int32),
        [pltpu.SemaphoreType.DMA] * 4,
    ],
)
def _hist_k(dst_hbm, degp_hbm, hist, zbuf, obuf, iall, hsems):
    c = lax.axis_index("c")
    s = lax.axis_index("s")
    wid = s * 2 + c
    zero16 = jnp.zeros((16,), jnp.float32)
    one16 = jnp.ones((16,), jnp.float32)
    for i in range(SLAB // 16):
        zbuf[pl.ds(i * 16, 16)] = zero16
    for i in range(B // 16):
        obuf[pl.ds(i * 16, 16)] = one16
    pltpu.sync_copy(zbuf, hist.at[pl.ds(s * SLAB, SLAB)])
    pltpu.sync_copy(dst_hbm.at[pl.ds(wid * HR, HR)], iall)
    plsc.subcore_barrier()

    def body(t, carry):
        for b in range(4):
            j = 4 * t + b

            @pl.when(t > 0)
            def _():
                pltpu.make_async_copy(obuf, hist.at[iall.at[0]],
                                      hsems[b]).wait()

            pltpu.async_copy(obuf, hist.at[iall.at[j]], hsems[b], add=True)
        return carry

    lax.fori_loop(0, HR // 4, body, 0)
    for b in range(4):
        pltpu.make_async_copy(obuf, hist.at[iall.at[0]], hsems[b]).wait()
    plsc.subcore_barrier()
    pltpu.sync_copy(hist.at[pl.ds(s * SLAB, SLAB)],
                    degp_hbm.at[pl.ds(c * NP + s * SLAB, SLAB)])


_NBUF = 2
_CH = MR // 2


@functools.partial(
    pl.kernel,
    out_type=jax.ShapeDtypeStruct((2 * NP, HALF), jnp.float32),
    mesh=_mesh,
    scratch_types=[
        pltpu.VMEM_SHARED((NP, HALF), jnp.float32),
        pltpu.VMEM((_CH, B), jnp.int32),
        pltpu.VMEM((_CH, B), jnp.int32),
        pltpu.VMEM((_NBUF, B, HALF), jnp.float32),
        [pltpu.SemaphoreType.DMA] * _NBUF,
        [pltpu.SemaphoreType.DMA] * _NBUF,
    ],
)
def _gs_k(y_hbm, src3_hbm, dst_hbm, out_hbm, acc, sall, dall, rbuf,
          gsems, ssems):
    c = lax.axis_index("c")
    s = lax.axis_index("s")
    base = s * MR

    def stage_idx(k):
        pltpu.sync_copy(src3_hbm.at[c, pl.ds(base + k * _CH, _CH)], sall)
        pltpu.sync_copy(dst_hbm.at[pl.ds(base + k * _CH, _CH)], dall)

    def start_gather(b, j):
        pltpu.async_copy(y_hbm.at[sall.at[j]], rbuf.at[b], gsems[b])

    def wait_gather(b):
        pltpu.make_async_copy(y_hbm.at[sall.at[0]], rbuf.at[b],
                              gsems[b]).wait()

    def start_scatter(b, j):
        pltpu.async_copy(rbuf.at[b], acc.at[dall.at[j]], ssems[b], add=True)

    def wait_scatter(b):
        pltpu.make_async_copy(rbuf.at[b], acc.at[dall.at[0]],
                              ssems[b]).wait()

    stage_idx(0)
    pltpu.sync_copy(y_hbm.at[pl.ds(c * NP + s * SLAB, SLAB)],
                    acc.at[pl.ds(s * SLAB, SLAB)])
    plsc.subcore_barrier()

    for k in range(MR // _CH):
        for b in range(_NBUF):
            start_gather(b, b)

        def body(t, carry):
            for b in range(_NBUF):
                j = _NBUF * t + b
                wait_gather(b)
                start_scatter(b, j)

                @pl.when(j < _CH - _NBUF)
                def _():
                    wait_scatter(b)
                    start_gather(b, j + _NBUF)

            return carry

        lax.fori_loop(0, _CH // _NBUF, body, 0)
        for b in range(_NBUF):
            wait_scatter(b)
        if k + 1 < MR // _CH:
            stage_idx(k + 1)

    plsc.subcore_barrier()
    pltpu.sync_copy(acc.at[pl.ds(s * SLAB, SLAB)],
                    out_hbm.at[pl.ds(c * NP + s * SLAB, SLAB)])


def _mm_scale_body(x_ref, w_ref, degp_ref, y_ref):
    d = degp_ref[...]
    deg = d[:, 0:1] + d[:, 1:2] + 1.0
    dis = lax.rsqrt(deg)
    xw = jnp.dot(x_ref[...], w_ref[...],
                 preferred_element_type=jnp.float32)
    y = xw * dis
    y_ref[0, :, :] = y[:, :HALF]
    y_ref[1, :, :] = y[:, HALF:]


def _epi_body(acc_ref, degp_ref, b0_ref, g_ref, be_ref, wc_ref, bc_ref, out_ref):
    d = degp_ref[...]
    deg = d[:, 0:1] + d[:, 1:2] + 1.0
    dis = lax.rsqrt(deg)
    h0 = jnp.maximum(acc_ref[0] * dis + b0_ref[:, :HALF], 0.0)
    h1 = jnp.maximum(acc_ref[1] * dis + b0_ref[:, HALF:], 0.0)
    mu = (jnp.sum(h0, axis=-1, keepdims=True)
          + jnp.sum(h1, axis=-1, keepdims=True)) * (1.0 / H)
    c0 = h0 - mu
    c1 = h1 - mu
    var = (jnp.sum(c0 * c0, axis=-1, keepdims=True)
           + jnp.sum(c1 * c1, axis=-1, keepdims=True)) * (1.0 / H)
    r = lax.rsqrt(var + 1e-5)
    n0 = c0 * r * g_ref[:, :HALF] + be_ref[:, :HALF]
    n1 = c1 * r * g_ref[:, HALF:] + be_ref[:, HALF:]
    out_ref[...] = (jnp.dot(n0, wc_ref[:HALF, :],
                            preferred_element_type=jnp.float32)
                    + jnp.dot(n1, wc_ref[HALF:, :],
                              preferred_element_type=jnp.float32)
                    + bc_ref[...])


_BN = 1000

_mm_call = pl.pallas_call(
    _mm_scale_body,
    grid=(N // _BN,),
    in_specs=[
        pl.BlockSpec((_BN, D), lambda j: (j, 0)),
        pl.BlockSpec((D, H), lambda j: (0, 0)),
        pl.BlockSpec((_BN, 2), lambda j: (j, 0)),
    ],
    out_specs=pl.BlockSpec((2, _BN, HALF), lambda j: (0, j, 0)),
    out_shape=jax.ShapeDtypeStruct((2, NP, HALF), jnp.float32),
)

_epi_call = pl.pallas_call(
    _epi_body,
    grid=(N // _BN,),
    in_specs=[
        pl.BlockSpec((2, _BN, HALF), lambda j: (0, j, 0)),
        pl.BlockSpec((_BN, 2), lambda j: (j, 0)),
        pl.BlockSpec((1, H), lambda j: (0, 0)),
        pl.BlockSpec((1, H), lambda j: (0, 0)),
        pl.BlockSpec((1, H), lambda j: (0, 0)),
        pl.BlockSpec((H, C), lambda j: (0, 0)),
        pl.BlockSpec((1, C), lambda j: (0, 0)),
    ],
    out_specs=pl.BlockSpec((_BN, C), lambda j: (j, 0)),
    out_shape=jax.ShapeDtypeStruct((N, C), jnp.float32),
)


def kernel(x, edge_index, W0, b0, gamma0, beta0, Wc, bc):
    src = edge_index[0]
    dst = edge_index[1]
    src_p = jnp.concatenate([src, jnp.zeros((PAD,), jnp.int32)])
    dst_p = jnp.concatenate([dst, jnp.full((PAD,), N, jnp.int32)])
    dst2d = dst_p.reshape(ROWS, B)
    src3 = jnp.stack([src_p, src_p + NP]).reshape(2, ROWS, B)

    degp_flat = _hist_k(dst2d)
    degp = jnp.stack([degp_flat[:NP], degp_flat[NP:]], axis=1)
    yst = _mm_call(x, W0, degp)
    y2 = yst.reshape(2 * NP, HALF)

    accf = _gs_k(y2, src3, dst2d)
    acc3 = accf.reshape(2, NP, HALF)

    return _epi_call(acc3, degp,
                     b0.reshape(1, H), gamma0.reshape(1, H),
                     beta0.reshape(1, H), Wc, bc.reshape(1, C))
